# baseline (speedup 1.0000x reference)
import jax
import jax.numpy as jnp
from jax import lax
from jax.experimental import pallas as pl
from jax.experimental.pallas import tpu as pltpu
from jax.experimental.pallas import tpu_sc as plsc

_N = 131072
_D = 64
_BC = 16384

_NUM_WORKERS = 32
_PER_W = _N // _NUM_WORKERS
_LANES = 16


def _sc_body(ct_hbm, dt_hbm, ntc_hbm, ntd_hbm, tc_v, td_v, ntc_v, ntd_v):
    wid = lax.axis_index("s") * 2 + lax.axis_index("c")
    base = wid * _PER_W
    pltpu.sync_copy(ct_hbm.at[0, pl.ds(base, _PER_W)], tc_v)
    pltpu.sync_copy(dt_hbm.at[0, pl.ds(base, _PER_W)], td_v)

    def step(i, _):
        sl = pl.ds(i * _LANES, _LANES)
        tcv = tc_v[sl]
        tdv = td_v[sl]
        straddle = ((tcv - tdv) < 0.0) & ((tcv + tdv) >= 0.0)
        ntc_v[sl] = jnp.where(straddle, 0.75 * tcv, tcv)
        ntd_v[sl] = jnp.where(straddle, 1.25 * tdv, tdv)
        return ()

    lax.fori_loop(0, _PER_W // _LANES, step, ())
    pltpu.sync_copy(ntc_v, ntc_hbm.at[pl.ds(base, _PER_W)])
    pltpu.sync_copy(ntd_v, ntd_hbm.at[pl.ds(base, _PER_W)])


_sc_branch = pl.kernel(
    _sc_body,
    out_type=(
        jax.ShapeDtypeStruct((_N,), jnp.float32),
        jax.ShapeDtypeStruct((_N,), jnp.float32),
    ),
    mesh=plsc.VectorSubcoreMesh(core_axis_name="c", subcore_axis_name="s"),
    scratch_types=[
        pltpu.VMEM((_PER_W,), jnp.float32),
        pltpu.VMEM((_PER_W,), jnp.float32),
        pltpu.VMEM((_PER_W,), jnp.float32),
        pltpu.VMEM((_PER_W,), jnp.float32),
    ],
    compiler_params=pltpu.CompilerParams(use_tc_tiling_on_sc=True),
)


def _tc_copy_body(c_ref, d_ref, out_ref):
    out_ref[0] = c_ref[...]
    out_ref[1] = d_ref[...]


def _tc_fix_body(ntc_ref, ntd_ref, prev_ref, out_ref):
    row0 = jax.lax.broadcasted_iota(jnp.int32, (8, 1), 0) == 0
    out_ref[0] = jnp.where(row0, ntc_ref[...].reshape(1, _BC), prev_ref[0])
    out_ref[1] = jnp.where(row0, ntd_ref[...].reshape(1, _BC), prev_ref[1])


def kernel(c, delta):
    ct = c.T
    dt = delta.T
    ntc, ntd = _sc_branch(ct, dt)
    out_c = pl.pallas_call(
        _tc_copy_body,
        grid=(_N // _BC,),
        in_specs=[
            pl.BlockSpec((_D, _BC), lambda i: (0, i)),
            pl.BlockSpec((_D, _BC), lambda i: (0, i)),
        ],
        out_specs=pl.BlockSpec((2, _D, _BC), lambda i: (0, 0, i)),
        out_shape=jax.ShapeDtypeStruct((2, _D, _N), jnp.float32),
        compiler_params=pltpu.CompilerParams(
            dimension_semantics=("arbitrary",),
        ),
    )(ct, dt)
    out_t = pl.pallas_call(
        _tc_fix_body,
        grid=(_N // _BC,),
        in_specs=[
            pl.BlockSpec((_BC,), lambda i: (i,)),
            pl.BlockSpec((_BC,), lambda i: (i,)),
            pl.BlockSpec((2, 8, _BC), lambda i: (0, 0, i)),
        ],
        out_specs=pl.BlockSpec((2, 8, _BC), lambda i: (0, 0, i)),
        out_shape=jax.ShapeDtypeStruct((2, _D, _N), jnp.float32),
        input_output_aliases={2: 0},
        compiler_params=pltpu.CompilerParams(
            dimension_semantics=("arbitrary",),
        ),
    )(ntc, ntd, out_c)
    return out_t.transpose(0, 2, 1)


# speedup vs baseline: 1.0537x; 1.0537x over previous
import jax
import jax.numpy as jnp
from jax import lax
from jax.experimental import pallas as pl
from jax.experimental.pallas import tpu as pltpu
from jax.experimental.pallas import tpu_sc as plsc

_N = 131072
_D = 64
_BC = 16384

_NUM_WORKERS = 32
_PER_W = _N // _NUM_WORKERS
_LANES = 16


def _sc_body(ct_hbm, dt_hbm, ntc_hbm, ntd_hbm, tc_v, td_v, ntc_v, ntd_v):
    wid = lax.axis_index("s") * 2 + lax.axis_index("c")
    base = wid * _PER_W
    pltpu.sync_copy(ct_hbm.at[0, pl.ds(base, _PER_W)], tc_v)
    pltpu.sync_copy(dt_hbm.at[0, pl.ds(base, _PER_W)], td_v)

    def step(i, _):
        sl = pl.ds(i * _LANES, _LANES)
        tcv = tc_v[sl]
        tdv = td_v[sl]
        straddle = ((tcv - tdv) < 0.0) & ((tcv + tdv) >= 0.0)
        ntc_v[sl] = jnp.where(straddle, 0.75 * tcv, tcv)
        ntd_v[sl] = jnp.where(straddle, 1.25 * tdv, tdv)
        return ()

    lax.fori_loop(0, _PER_W // _LANES, step, ())
    pltpu.sync_copy(ntc_v, ntc_hbm.at[pl.ds(base, _PER_W)])
    pltpu.sync_copy(ntd_v, ntd_hbm.at[pl.ds(base, _PER_W)])


_sc_branch = pl.kernel(
    _sc_body,
    out_type=(
        jax.ShapeDtypeStruct((_N,), jnp.float32),
        jax.ShapeDtypeStruct((_N,), jnp.float32),
    ),
    mesh=plsc.VectorSubcoreMesh(core_axis_name="c", subcore_axis_name="s"),
    scratch_types=[
        pltpu.VMEM((_PER_W,), jnp.float32),
        pltpu.VMEM((_PER_W,), jnp.float32),
        pltpu.VMEM((_PER_W,), jnp.float32),
        pltpu.VMEM((_PER_W,), jnp.float32),
    ],
    compiler_params=pltpu.CompilerParams(
        use_tc_tiling_on_sc=True,
        skip_device_barrier=True,
    ),
)


def _tc_body(c_ref, d_ref, ntc_ref, ntd_ref, out_ref):
    cv = c_ref[...]
    dv = d_ref[...]
    row0 = jax.lax.broadcasted_iota(jnp.int32, (_D, 1), 0) == 0
    out_ref[0] = jnp.where(row0, ntc_ref[...].reshape(1, _BC), cv)
    out_ref[1] = jnp.where(row0, ntd_ref[...].reshape(1, _BC), dv)


def kernel(c, delta):
    ct = c.T
    dt = delta.T
    ntc, ntd = _sc_branch(ct, dt)
    out_t = pl.pallas_call(
        _tc_body,
        grid=(_N // _BC,),
        in_specs=[
            pl.BlockSpec((_D, _BC), lambda i: (0, i)),
            pl.BlockSpec((_D, _BC), lambda i: (0, i)),
            pl.BlockSpec((_BC,), lambda i: (i,)),
            pl.BlockSpec((_BC,), lambda i: (i,)),
        ],
        out_specs=pl.BlockSpec((2, _D, _BC), lambda i: (0, 0, i)),
        out_shape=jax.ShapeDtypeStruct((2, _D, _N), jnp.float32),
        compiler_params=pltpu.CompilerParams(
            dimension_semantics=("arbitrary",),
        ),
    )(ct, dt, ntc, ntd)
    return out_t.transpose(0, 2, 1)


# speedup vs baseline: 1.0612x; 1.0071x over previous
import jax
import jax.numpy as jnp
from jax import lax
from jax.experimental import pallas as pl
from jax.experimental.pallas import tpu as pltpu
from jax.experimental.pallas import tpu_sc as plsc

_N = 131072
_D = 64
_BC = 16384

_NUM_WORKERS = 32
_PER_W = _N // _NUM_WORKERS
_LANES = 16


def _sc_body(ct_hbm, dt_hbm, ntc_hbm, ntd_hbm, tc_v, td_v, ntc_v, ntd_v):
    wid = lax.axis_index("s") * 2 + lax.axis_index("c")
    base = wid * _PER_W
    pltpu.sync_copy(ct_hbm.at[0, pl.ds(base, _PER_W)], tc_v)
    pltpu.sync_copy(dt_hbm.at[0, pl.ds(base, _PER_W)], td_v)

    def step(i, _):
        for k in range(8):
            sl = pl.ds((i * 8 + k) * _LANES, _LANES)
            tcv = tc_v[sl]
            tdv = td_v[sl]
            straddle = ((tcv - tdv) < 0.0) & ((tcv + tdv) >= 0.0)
            ntc_v[sl] = jnp.where(straddle, 0.75 * tcv, tcv)
            ntd_v[sl] = jnp.where(straddle, 1.25 * tdv, tdv)
        return ()

    lax.fori_loop(0, _PER_W // (_LANES * 8), step, ())
    pltpu.sync_copy(ntc_v, ntc_hbm.at[pl.ds(base, _PER_W)])
    pltpu.sync_copy(ntd_v, ntd_hbm.at[pl.ds(base, _PER_W)])


_sc_branch = pl.kernel(
    _sc_body,
    out_type=(
        jax.ShapeDtypeStruct((_N,), jnp.float32),
        jax.ShapeDtypeStruct((_N,), jnp.float32),
    ),
    mesh=plsc.VectorSubcoreMesh(core_axis_name="c", subcore_axis_name="s"),
    scratch_types=[
        pltpu.VMEM((_PER_W,), jnp.float32),
        pltpu.VMEM((_PER_W,), jnp.float32),
        pltpu.VMEM((_PER_W,), jnp.float32),
        pltpu.VMEM((_PER_W,), jnp.float32),
    ],
    compiler_params=pltpu.CompilerParams(
        use_tc_tiling_on_sc=True,
        skip_device_barrier=True,
    ),
)


def _tc_body(c_ref, d_ref, ntc_ref, ntd_ref, out_ref):
    cv = c_ref[...]
    dv = d_ref[...]
    row0 = jax.lax.broadcasted_iota(jnp.int32, (_D, 1), 0) == 0
    out_ref[0] = jnp.where(row0, ntc_ref[...].reshape(1, _BC), cv)
    out_ref[1] = jnp.where(row0, ntd_ref[...].reshape(1, _BC), dv)


def kernel(c, delta):
    ct = c.T
    dt = delta.T
    ntc, ntd = _sc_branch(ct, dt)
    out_t = pl.pallas_call(
        _tc_body,
        grid=(_N // _BC,),
        in_specs=[
            pl.BlockSpec((_D, _BC), lambda i: (0, i)),
            pl.BlockSpec((_D, _BC), lambda i: (0, i)),
            pl.BlockSpec((_BC,), lambda i: (i,)),
            pl.BlockSpec((_BC,), lambda i: (i,)),
        ],
        out_specs=pl.BlockSpec((2, _D, _BC), lambda i: (0, 0, i)),
        out_shape=jax.ShapeDtypeStruct((2, _D, _N), jnp.float32),
        compiler_params=pltpu.CompilerParams(
            dimension_semantics=("arbitrary",),
        ),
    )(ct, dt, ntc, ntd)
    return out_t.transpose(0, 2, 1)
